# Initial kernel scaffold; baseline (speedup 1.0000x reference)
import jax
import jax.numpy as jnp
from jax.experimental import pallas as pl


def kernel(edges, features, W1, b1, W2, b2, W3, b3):
    raise NotImplementedError("write your pallas kernel here")


# speedup vs baseline: 12.0441x; 12.0441x over previous
import functools

import jax
import jax.numpy as jnp
from jax import lax
from jax.experimental import pallas as pl
from jax.experimental.pallas import tpu as pltpu
from jax.experimental.pallas import tpu_sc as plsc

N, E, F_IN, H = 10000, 320000, 128, 64
NC, NS, L = 2, 16, 16
NW = NC * NS
NPAD = 10240
EPW = E // NW
EW = 10240
K = 128
CH = EW // K
RPT = NPAD // NS
RB = 1024
GRID = NPAD // RB

_MESH = plsc.VectorSubcoreMesh(
    core_axis_name="c", subcore_axis_name="s", num_cores=NC, num_subcores=NS
)


def _deg_body(dsts_hbm, degp_hbm, idx_v, degbuf):
    c = lax.axis_index("c")
    s = lax.axis_index("s")
    wid = s * NC + c
    pltpu.sync_copy(dsts_hbm.at[wid], idx_v)

    zeros = jnp.zeros((L,), jnp.float32)

    @pl.loop(0, NPAD // L, unroll=8)
    def _(i):
        degbuf[pl.ds(i * L, L)] = zeros

    ones = jnp.ones((L,), jnp.float32)

    @pl.loop(0, EW // L, unroll=8)
    def _(i):
        idx = idx_v[pl.ds(i * L, L)]
        plsc.addupdate_scatter(degbuf, [idx], ones)

    pltpu.sync_copy(degbuf, degp_hbm.at[wid])


_SC_PARAMS = pltpu.CompilerParams(
    needs_layout_passes=False, use_tc_tiling_on_sc=False
)

_deg_call = functools.partial(
    pl.kernel,
    out_type=jax.ShapeDtypeStruct((NW, NPAD), jnp.float32),
    mesh=_MESH,
    compiler_params=_SC_PARAMS,
    scratch_types=[
        pltpu.VMEM((EW,), jnp.int32),
        pltpu.VMEM((NPAD,), jnp.float32),
    ],
)(_deg_body)


def _prop_body(hp_hbm, srcs_hbm, dsts_hbm, out_hbm, idxs_v, idxd_v, gbuf, acc, sem):
    c = lax.axis_index("c")
    s = lax.axis_index("s")
    wid = s * NC + c
    pltpu.sync_copy(srcs_hbm.at[wid], idxs_v)
    pltpu.sync_copy(dsts_hbm.at[wid], idxd_v)

    zeros = jnp.zeros((L,), jnp.float32)

    @pl.loop(0, K, unroll=4)
    def _(i):
        for j in range(H // L):
            gbuf[0, i, pl.ds(j * L, L)] = zeros

    for t in range(RPT // K):
        pltpu.sync_copy(gbuf.at[0], acc.at[pl.ds(s * RPT + t * K, K)])

    plsc.subcore_barrier()

    def _gather(j, b):
        pltpu.async_copy(hp_hbm.at[idxs_v.at[j]], gbuf.at[b], sem)

    def _wait():
        pltpu.make_async_copy(hp_hbm.at[pl.ds(0, K)], gbuf.at[0], sem).wait()

    def _scatter(j, b):
        pltpu.sync_copy(gbuf.at[b], acc.at[idxd_v.at[j]], add=True)

    _gather(0, 0)

    @pl.loop(0, CH // 2)
    def _(j2):
        j = j2 * 2
        _gather(j + 1, 1)
        _wait()
        _scatter(j, 0)
        _gather(j + 2, 0)
        _wait()
        _scatter(j + 1, 1)

    _wait()
    plsc.subcore_barrier()
    pltpu.sync_copy(
        acc.at[pl.ds(s * RPT, RPT)], out_hbm.at[c].at[pl.ds(s * RPT, RPT)]
    )


_prop_call = functools.partial(
    pl.kernel,
    out_type=jax.ShapeDtypeStruct((NC, NPAD, H), jnp.float32),
    mesh=_MESH,
    compiler_params=_SC_PARAMS,
    scratch_types=[
        pltpu.VMEM((CH + 1, K), jnp.int32),
        pltpu.VMEM((CH, K), jnp.int32),
        pltpu.VMEM((2, K, H), jnp.float32),
        pltpu.VMEM_SHARED((NPAD, H), jnp.float32),
        pltpu.SemaphoreType.DMA,
    ],
)(_prop_body)


def _tc1_body(x_ref, w_ref, degp_ref, hp_ref, dis_ref):
    deg = jnp.sum(degp_ref[...], axis=0) + 1.0
    dis = lax.rsqrt(deg)[:, None]
    h = jnp.dot(x_ref[...], w_ref[...], preferred_element_type=jnp.float32)
    hp_ref[...] = h * dis
    dis_ref[...] = dis


def _tc1_call(xpad, w1, degp):
    return pl.pallas_call(
        _tc1_body,
        grid=(GRID,),
        in_specs=[
            pl.BlockSpec((RB, F_IN), lambda i: (i, 0)),
            pl.BlockSpec((F_IN, H), lambda i: (0, 0)),
            pl.BlockSpec((NW, RB), lambda i: (0, i)),
        ],
        out_specs=[
            pl.BlockSpec((RB, H), lambda i: (i, 0)),
            pl.BlockSpec((RB, 1), lambda i: (i, 0)),
        ],
        out_shape=[
            jax.ShapeDtypeStruct((NPAD, H), jnp.float32),
            jax.ShapeDtypeStruct((NPAD, 1), jnp.float32),
        ],
    )(xpad, w1, degp)


def _tc2_body(accp_ref, hp_ref, dis_ref, b_ref, w_ref, out_ref):
    t = accp_ref[0] + accp_ref[1] + hp_ref[...]
    t = t * dis_ref[...] + b_ref[...]
    t = jnp.maximum(t, 0.0)
    h2 = jnp.dot(t, w_ref[...], preferred_element_type=jnp.float32)
    out_ref[...] = h2 * dis_ref[...]


def _tc2_call(accp, hp, dis, b2d, w):
    return pl.pallas_call(
        _tc2_body,
        grid=(GRID,),
        in_specs=[
            pl.BlockSpec((NC, RB, H), lambda i: (0, i, 0)),
            pl.BlockSpec((RB, H), lambda i: (i, 0)),
            pl.BlockSpec((RB, 1), lambda i: (i, 0)),
            pl.BlockSpec((1, H), lambda i: (0, 0)),
            pl.BlockSpec((H, H), lambda i: (0, 0)),
        ],
        out_specs=pl.BlockSpec((RB, H), lambda i: (i, 0)),
        out_shape=jax.ShapeDtypeStruct((NPAD, H), jnp.float32),
    )(accp, hp, dis, b2d, w)


def _tc3_body(accp_ref, hp_ref, dis_ref, b_ref, out_ref):
    z = accp_ref[0] + accp_ref[1] + hp_ref[...]
    z = z * dis_ref[...] + b_ref[...]
    m = jnp.max(z, axis=1, keepdims=True)
    lse = jnp.log(jnp.sum(jnp.exp(z - m), axis=1, keepdims=True)) + m
    out_ref[...] = z - lse


def _tc3_call(accp, hp, dis, b2d):
    return pl.pallas_call(
        _tc3_body,
        grid=(GRID,),
        in_specs=[
            pl.BlockSpec((NC, RB, H), lambda i: (0, i, 0)),
            pl.BlockSpec((RB, H), lambda i: (i, 0)),
            pl.BlockSpec((RB, 1), lambda i: (i, 0)),
            pl.BlockSpec((1, H), lambda i: (0, 0)),
        ],
        out_specs=pl.BlockSpec((RB, H), lambda i: (i, 0)),
        out_shape=jax.ShapeDtypeStruct((N, H), jnp.float32),
    )(accp, hp, dis, b2d)


def kernel(edges, features, W1, b1, W2, b2, W3, b3):
    src = edges[0].astype(jnp.int32)
    dst = edges[1].astype(jnp.int32)
    pad = jnp.full((NW, EW - EPW), N, jnp.int32)
    srcw = jnp.concatenate([src.reshape(NW, EPW), pad], axis=1)
    dstw = jnp.concatenate([dst.reshape(NW, EPW), pad], axis=1)
    srcs = jnp.concatenate(
        [srcw.reshape(NW, CH, K), jnp.full((NW, 1, K), N, jnp.int32)], axis=1
    )
    dsts = dstw.reshape(NW, CH, K)

    degp = _deg_call(dstw)
    xpad = jnp.pad(features, ((0, NPAD - N), (0, 0)))
    hp1, dis = _tc1_call(xpad, W1, degp)
    acc1 = _prop_call(hp1, srcs, dsts)
    hp2 = _tc2_call(acc1, hp1, dis, b1.reshape(1, H), W2)
    acc2 = _prop_call(hp2, srcs, dsts)
    hp3 = _tc2_call(acc2, hp2, dis, b2.reshape(1, H), W3)
    acc3 = _prop_call(hp3, srcs, dsts)
    return _tc3_call(acc3, hp3, dis, b3.reshape(1, H))


# speedup vs baseline: 15.6218x; 1.2970x over previous
import functools

import jax
import jax.numpy as jnp
from jax import lax
from jax.experimental import pallas as pl
from jax.experimental.pallas import tpu as pltpu
from jax.experimental.pallas import tpu_sc as plsc

N, E, F_IN, H = 10000, 320000, 128, 64
NC, NS, L = 2, 16, 16
NW = NC * NS
NPAD = 10240
EPW = E // NW
EW = 10240
K = 128
CH = EW // K
RPT = NPAD // NS
RB = 1024
GRID = NPAD // RB

_MESH = plsc.VectorSubcoreMesh(
    core_axis_name="c", subcore_axis_name="s", num_cores=NC, num_subcores=NS
)


def _deg_body(dsts_hbm, degp_hbm, idx_v, degbuf):
    c = lax.axis_index("c")
    s = lax.axis_index("s")
    wid = s * NC + c
    pltpu.sync_copy(dsts_hbm.at[wid], idx_v)

    zeros = jnp.zeros((L,), jnp.float32)

    @pl.loop(0, NPAD // L, unroll=8)
    def _(i):
        degbuf[pl.ds(i * L, L)] = zeros

    ones = jnp.ones((L,), jnp.float32)

    @pl.loop(0, EW // L, unroll=8)
    def _(i):
        idx = idx_v[pl.ds(i * L, L)]
        plsc.addupdate_scatter(degbuf, [idx], ones)

    pltpu.sync_copy(degbuf, degp_hbm.at[wid])


_SC_PARAMS = pltpu.CompilerParams(
    needs_layout_passes=False, use_tc_tiling_on_sc=False
)

_deg_call = functools.partial(
    pl.kernel,
    out_type=jax.ShapeDtypeStruct((NW, NPAD), jnp.float32),
    mesh=_MESH,
    compiler_params=_SC_PARAMS,
    scratch_types=[
        pltpu.VMEM((EW,), jnp.int32),
        pltpu.VMEM((NPAD,), jnp.float32),
    ],
)(_deg_body)


def _prop_body(hp_hbm, srcs_hbm, dsts_hbm, out_hbm, idxs_v, idxd_v, gbuf, acc,
               semg, sems):
    c = lax.axis_index("c")
    s = lax.axis_index("s")
    wid = s * NC + c
    pltpu.sync_copy(srcs_hbm.at[wid], idxs_v)
    pltpu.sync_copy(dsts_hbm.at[wid], idxd_v)

    zeros = jnp.zeros((L,), jnp.float32)

    @pl.loop(0, K, unroll=4)
    def _(i):
        for j in range(H // L):
            gbuf[0, i, pl.ds(j * L, L)] = zeros

    for t in range(RPT // K):
        pltpu.sync_copy(gbuf.at[0], acc.at[pl.ds(s * RPT + t * K, K)])

    plsc.subcore_barrier()

    def _g(p, base):
        pltpu.async_copy(hp_hbm.at[idxs_v.at[2 * p]], gbuf.at[base], semg)
        pltpu.async_copy(hp_hbm.at[idxs_v.at[2 * p + 1]], gbuf.at[base + 1], semg)

    def _s(p, base):
        pltpu.async_copy(gbuf.at[base], acc.at[idxd_v.at[2 * p]], sems, add=True)
        pltpu.async_copy(
            gbuf.at[base + 1], acc.at[idxd_v.at[2 * p + 1]], sems, add=True
        )

    def _wg():
        pltpu.make_async_copy(hp_hbm.at[pl.ds(0, K)], gbuf.at[0], semg).wait()
        pltpu.make_async_copy(hp_hbm.at[pl.ds(0, K)], gbuf.at[0], semg).wait()

    def _ws():
        pltpu.make_async_copy(gbuf.at[0], acc.at[pl.ds(0, K)], sems).wait()
        pltpu.make_async_copy(gbuf.at[0], acc.at[pl.ds(0, K)], sems).wait()

    NP = CH // 2
    _g(0, 0)
    _wg()
    _s(0, 0)
    _g(1, 2)

    @pl.loop(0, (NP - 2) // 2)
    def _(q):
        p = 2 * q + 1
        _wg()
        _s(p, 2)
        _ws()
        _g(p + 1, 0)
        _wg()
        _s(p + 1, 0)
        _ws()
        _g(p + 2, 2)

    _wg()
    _s(NP - 1, 2)
    _ws()
    _ws()
    plsc.subcore_barrier()
    pltpu.sync_copy(
        acc.at[pl.ds(s * RPT, RPT)], out_hbm.at[c].at[pl.ds(s * RPT, RPT)]
    )


_prop_call = functools.partial(
    pl.kernel,
    out_type=jax.ShapeDtypeStruct((NC, NPAD, H), jnp.float32),
    mesh=_MESH,
    compiler_params=_SC_PARAMS,
    scratch_types=[
        pltpu.VMEM((CH, K), jnp.int32),
        pltpu.VMEM((CH, K), jnp.int32),
        pltpu.VMEM((4, K, H), jnp.float32),
        pltpu.VMEM_SHARED((NPAD, H), jnp.float32),
        pltpu.SemaphoreType.DMA,
        pltpu.SemaphoreType.DMA,
    ],
)(_prop_body)


def _tc1_body(x_ref, w_ref, degp_ref, hp_ref, dis_ref):
    deg = jnp.sum(degp_ref[...], axis=0) + 1.0
    dis = lax.rsqrt(deg)[:, None]
    h = jnp.dot(x_ref[...], w_ref[...], preferred_element_type=jnp.float32)
    hp_ref[...] = h * dis
    dis_ref[...] = dis


def _tc1_call(xpad, w1, degp):
    return pl.pallas_call(
        _tc1_body,
        grid=(GRID,),
        in_specs=[
            pl.BlockSpec((RB, F_IN), lambda i: (i, 0)),
            pl.BlockSpec((F_IN, H), lambda i: (0, 0)),
            pl.BlockSpec((NW, RB), lambda i: (0, i)),
        ],
        out_specs=[
            pl.BlockSpec((RB, H), lambda i: (i, 0)),
            pl.BlockSpec((RB, 1), lambda i: (i, 0)),
        ],
        out_shape=[
            jax.ShapeDtypeStruct((NPAD, H), jnp.float32),
            jax.ShapeDtypeStruct((NPAD, 1), jnp.float32),
        ],
    )(xpad, w1, degp)


def _tc2_body(accp_ref, hp_ref, dis_ref, b_ref, w_ref, out_ref):
    t = accp_ref[0] + accp_ref[1] + hp_ref[...]
    t = t * dis_ref[...] + b_ref[...]
    t = jnp.maximum(t, 0.0)
    h2 = jnp.dot(t, w_ref[...], preferred_element_type=jnp.float32)
    out_ref[...] = h2 * dis_ref[...]


def _tc2_call(accp, hp, dis, b2d, w):
    return pl.pallas_call(
        _tc2_body,
        grid=(GRID,),
        in_specs=[
            pl.BlockSpec((NC, RB, H), lambda i: (0, i, 0)),
            pl.BlockSpec((RB, H), lambda i: (i, 0)),
            pl.BlockSpec((RB, 1), lambda i: (i, 0)),
            pl.BlockSpec((1, H), lambda i: (0, 0)),
            pl.BlockSpec((H, H), lambda i: (0, 0)),
        ],
        out_specs=pl.BlockSpec((RB, H), lambda i: (i, 0)),
        out_shape=jax.ShapeDtypeStruct((NPAD, H), jnp.float32),
    )(accp, hp, dis, b2d, w)


def _tc3_body(accp_ref, hp_ref, dis_ref, b_ref, out_ref):
    z = accp_ref[0] + accp_ref[1] + hp_ref[...]
    z = z * dis_ref[...] + b_ref[...]
    m = jnp.max(z, axis=1, keepdims=True)
    lse = jnp.log(jnp.sum(jnp.exp(z - m), axis=1, keepdims=True)) + m
    out_ref[...] = z - lse


def _tc3_call(accp, hp, dis, b2d):
    return pl.pallas_call(
        _tc3_body,
        grid=(GRID,),
        in_specs=[
            pl.BlockSpec((NC, RB, H), lambda i: (0, i, 0)),
            pl.BlockSpec((RB, H), lambda i: (i, 0)),
            pl.BlockSpec((RB, 1), lambda i: (i, 0)),
            pl.BlockSpec((1, H), lambda i: (0, 0)),
        ],
        out_specs=pl.BlockSpec((RB, H), lambda i: (i, 0)),
        out_shape=jax.ShapeDtypeStruct((N, H), jnp.float32),
    )(accp, hp, dis, b2d)


def kernel(edges, features, W1, b1, W2, b2, W3, b3):
    src = edges[0].astype(jnp.int32)
    dst = edges[1].astype(jnp.int32)
    pad = jnp.full((NW, EW - EPW), N, jnp.int32)
    srcw = jnp.concatenate([src.reshape(NW, EPW), pad], axis=1)
    dstw = jnp.concatenate([dst.reshape(NW, EPW), pad], axis=1)
    srcs = srcw.reshape(NW, CH, K)
    dsts = dstw.reshape(NW, CH, K)

    degp = _deg_call(dstw)
    xpad = jnp.pad(features, ((0, NPAD - N), (0, 0)))
    hp1, dis = _tc1_call(xpad, W1, degp)
    acc1 = _prop_call(hp1, srcs, dsts)
    hp2 = _tc2_call(acc1, hp1, dis, b1.reshape(1, H), W2)
    acc2 = _prop_call(hp2, srcs, dsts)
    hp3 = _tc2_call(acc2, hp2, dis, b2.reshape(1, H), W3)
    acc3 = _prop_call(hp3, srcs, dsts)
    return _tc3_call(acc3, hp3, dis, b3.reshape(1, H))


# speedup vs baseline: 16.5611x; 1.0601x over previous
import functools

import jax
import jax.numpy as jnp
from jax import lax
from jax.experimental import pallas as pl
from jax.experimental.pallas import tpu as pltpu
from jax.experimental.pallas import tpu_sc as plsc

N, E, F_IN, H = 10000, 320000, 128, 64
NC, NS, L = 2, 16, 16
NW = NC * NS
NPAD = 10240
EPW = E // NW
EW = 10240
K = 128
CH = EW // K
RPT = NPAD // NS
RB = 1024
GRID = NPAD // RB

_MESH = plsc.VectorSubcoreMesh(
    core_axis_name="c", subcore_axis_name="s", num_cores=NC, num_subcores=NS
)


def _deg_body(dsts_hbm, degp_hbm, idx_v, degbuf):
    c = lax.axis_index("c")
    s = lax.axis_index("s")
    wid = s * NC + c
    pltpu.sync_copy(dsts_hbm.at[wid], idx_v)

    zeros = jnp.zeros((L,), jnp.float32)

    @pl.loop(0, NPAD // L, unroll=8)
    def _(i):
        degbuf[pl.ds(i * L, L)] = zeros

    ones = jnp.ones((L,), jnp.float32)

    @pl.loop(0, EW // L, unroll=8)
    def _(i):
        idx = idx_v[pl.ds(i * L, L)]
        plsc.addupdate_scatter(degbuf, [idx], ones)

    pltpu.sync_copy(degbuf, degp_hbm.at[wid])


_SC_PARAMS = pltpu.CompilerParams(
    needs_layout_passes=False, use_tc_tiling_on_sc=False
)

_deg_call = functools.partial(
    pl.kernel,
    out_type=jax.ShapeDtypeStruct((NW, NPAD), jnp.float32),
    mesh=_MESH,
    compiler_params=_SC_PARAMS,
    scratch_types=[
        pltpu.VMEM((EW,), jnp.int32),
        pltpu.VMEM((NPAD,), jnp.float32),
    ],
)(_deg_body)


def _prop_body(hp_hbm, srcs_hbm, dsts_hbm, out_hbm, idxs_v, idxd_v, gbuf, acc,
               semg, sems):
    c = lax.axis_index("c")
    s = lax.axis_index("s")
    wid = s * NC + c
    pltpu.sync_copy(srcs_hbm.at[wid], idxs_v)
    pltpu.sync_copy(dsts_hbm.at[wid], idxd_v)

    zeros = jnp.zeros((L,), jnp.float32)

    @pl.loop(0, K, unroll=4)
    def _(i):
        for j in range(H // L):
            gbuf[0, i, pl.ds(j * L, L)] = zeros

    for t in range(RPT // K):
        pltpu.sync_copy(gbuf.at[0], acc.at[pl.ds(s * RPT + t * K, K)])

    plsc.subcore_barrier()

    def _g(p, base):
        pltpu.async_copy(hp_hbm.at[idxs_v.at[2 * p]], gbuf.at[base], semg)
        pltpu.async_copy(hp_hbm.at[idxs_v.at[2 * p + 1]], gbuf.at[base + 1], semg)

    def _s(p, base):
        pltpu.async_copy(gbuf.at[base], acc.at[idxd_v.at[2 * p]], sems, add=True)
        pltpu.async_copy(
            gbuf.at[base + 1], acc.at[idxd_v.at[2 * p + 1]], sems, add=True
        )

    def _wg():
        pltpu.make_async_copy(hp_hbm.at[pl.ds(0, K)], gbuf.at[0], semg).wait()
        pltpu.make_async_copy(hp_hbm.at[pl.ds(0, K)], gbuf.at[0], semg).wait()

    def _ws():
        pltpu.make_async_copy(gbuf.at[0], acc.at[pl.ds(0, K)], sems).wait()
        pltpu.make_async_copy(gbuf.at[0], acc.at[pl.ds(0, K)], sems).wait()

    NP = CH // 2
    _g(0, 0)
    _g(1, 2)
    _wg()
    _s(0, 0)
    _g(2, 4)
    _wg()
    _s(1, 2)
    _g(3, 6)

    @pl.loop(0, (NP - 4) // 4)
    def _(q):
        for k in range(4):
            p = 4 * q + 2 + k
            slot = (2 + k) % 4
            _wg()
            _s(p, 2 * slot)
            _ws()
            _g(p + 2, 2 * ((slot + 2) % 4))

    _wg()
    _s(NP - 2, 4)
    _ws()
    _wg()
    _s(NP - 1, 6)
    _ws()
    _ws()
    plsc.subcore_barrier()
    pltpu.sync_copy(
        acc.at[pl.ds(s * RPT, RPT)], out_hbm.at[c].at[pl.ds(s * RPT, RPT)]
    )


_prop_call = functools.partial(
    pl.kernel,
    out_type=jax.ShapeDtypeStruct((NC, NPAD, H), jnp.float32),
    mesh=_MESH,
    compiler_params=_SC_PARAMS,
    scratch_types=[
        pltpu.VMEM((CH, K), jnp.int32),
        pltpu.VMEM((CH, K), jnp.int32),
        pltpu.VMEM((8, K, H), jnp.float32),
        pltpu.VMEM_SHARED((NPAD, H), jnp.float32),
        pltpu.SemaphoreType.DMA,
        pltpu.SemaphoreType.DMA,
    ],
)(_prop_body)


def _tc1_body(x_ref, w_ref, degp_ref, hp_ref, dis_ref):
    deg = jnp.sum(degp_ref[...], axis=0) + 1.0
    dis = lax.rsqrt(deg)[:, None]
    h = jnp.dot(x_ref[...], w_ref[...], preferred_element_type=jnp.float32)
    hp_ref[...] = h * dis
    dis_ref[...] = dis


def _tc1_call(xpad, w1, degp):
    return pl.pallas_call(
        _tc1_body,
        grid=(GRID,),
        in_specs=[
            pl.BlockSpec((RB, F_IN), lambda i: (i, 0)),
            pl.BlockSpec((F_IN, H), lambda i: (0, 0)),
            pl.BlockSpec((NW, RB), lambda i: (0, i)),
        ],
        out_specs=[
            pl.BlockSpec((RB, H), lambda i: (i, 0)),
            pl.BlockSpec((RB, 1), lambda i: (i, 0)),
        ],
        out_shape=[
            jax.ShapeDtypeStruct((NPAD, H), jnp.float32),
            jax.ShapeDtypeStruct((NPAD, 1), jnp.float32),
        ],
    )(xpad, w1, degp)


def _tc2_body(accp_ref, hp_ref, dis_ref, b_ref, w_ref, out_ref):
    t = accp_ref[0] + accp_ref[1] + hp_ref[...]
    t = t * dis_ref[...] + b_ref[...]
    t = jnp.maximum(t, 0.0)
    h2 = jnp.dot(t, w_ref[...], preferred_element_type=jnp.float32)
    out_ref[...] = h2 * dis_ref[...]


def _tc2_call(accp, hp, dis, b2d, w):
    return pl.pallas_call(
        _tc2_body,
        grid=(GRID,),
        in_specs=[
            pl.BlockSpec((NC, RB, H), lambda i: (0, i, 0)),
            pl.BlockSpec((RB, H), lambda i: (i, 0)),
            pl.BlockSpec((RB, 1), lambda i: (i, 0)),
            pl.BlockSpec((1, H), lambda i: (0, 0)),
            pl.BlockSpec((H, H), lambda i: (0, 0)),
        ],
        out_specs=pl.BlockSpec((RB, H), lambda i: (i, 0)),
        out_shape=jax.ShapeDtypeStruct((NPAD, H), jnp.float32),
    )(accp, hp, dis, b2d, w)


def _tc3_body(accp_ref, hp_ref, dis_ref, b_ref, out_ref):
    z = accp_ref[0] + accp_ref[1] + hp_ref[...]
    z = z * dis_ref[...] + b_ref[...]
    m = jnp.max(z, axis=1, keepdims=True)
    lse = jnp.log(jnp.sum(jnp.exp(z - m), axis=1, keepdims=True)) + m
    out_ref[...] = z - lse


def _tc3_call(accp, hp, dis, b2d):
    return pl.pallas_call(
        _tc3_body,
        grid=(GRID,),
        in_specs=[
            pl.BlockSpec((NC, RB, H), lambda i: (0, i, 0)),
            pl.BlockSpec((RB, H), lambda i: (i, 0)),
            pl.BlockSpec((RB, 1), lambda i: (i, 0)),
            pl.BlockSpec((1, H), lambda i: (0, 0)),
        ],
        out_specs=pl.BlockSpec((RB, H), lambda i: (i, 0)),
        out_shape=jax.ShapeDtypeStruct((N, H), jnp.float32),
    )(accp, hp, dis, b2d)


def kernel(edges, features, W1, b1, W2, b2, W3, b3):
    src = edges[0].astype(jnp.int32)
    dst = edges[1].astype(jnp.int32)
    pad = jnp.full((NW, EW - EPW), N, jnp.int32)
    srcw = jnp.concatenate([src.reshape(NW, EPW), pad], axis=1)
    dstw = jnp.concatenate([dst.reshape(NW, EPW), pad], axis=1)
    srcs = srcw.reshape(NW, CH, K)
    dsts = dstw.reshape(NW, CH, K)

    degp = _deg_call(dstw)
    xpad = jnp.pad(features, ((0, NPAD - N), (0, 0)))
    hp1, dis = _tc1_call(xpad, W1, degp)
    acc1 = _prop_call(hp1, srcs, dsts)
    hp2 = _tc2_call(acc1, hp1, dis, b1.reshape(1, H), W2)
    acc2 = _prop_call(hp2, srcs, dsts)
    hp3 = _tc2_call(acc2, hp2, dis, b2.reshape(1, H), W3)
    acc3 = _prop_call(hp3, srcs, dsts)
    return _tc3_call(acc3, hp3, dis, b3.reshape(1, H))


# speedup vs baseline: 16.5648x; 1.0002x over previous
import functools

import jax
import jax.numpy as jnp
from jax import lax
from jax.experimental import pallas as pl
from jax.experimental.pallas import tpu as pltpu
from jax.experimental.pallas import tpu_sc as plsc

N, E, F_IN, H = 10000, 320000, 128, 64
NC, NS, L = 2, 16, 16
NW = NC * NS
NPAD = 10240
EPW = E // NW
EW = 10240
K = 128
CH = EW // K
RPT = NPAD // NS
RB = 1024
GRID = NPAD // RB

_MESH = plsc.VectorSubcoreMesh(
    core_axis_name="c", subcore_axis_name="s", num_cores=NC, num_subcores=NS
)

_SC_PARAMS = pltpu.CompilerParams(
    needs_layout_passes=False, use_tc_tiling_on_sc=False
)


def _deg_body(dsts_hbm, degp_hbm, idx_v, degbuf):
    c = lax.axis_index("c")
    s = lax.axis_index("s")
    wid = s * NC + c
    pltpu.sync_copy(dsts_hbm.at[wid], idx_v)

    zeros = jnp.zeros((L,), jnp.float32)

    @pl.loop(0, NPAD // L, unroll=8)
    def _(i):
        degbuf[pl.ds(i * L, L)] = zeros

    ones = jnp.ones((L,), jnp.float32)

    @pl.loop(0, EW // L, unroll=8)
    def _(i):
        idx = idx_v[pl.ds(i * L, L)]
        plsc.addupdate_scatter(degbuf, [idx], ones)

    pltpu.sync_copy(degbuf, degp_hbm.at[wid])


_deg_call = functools.partial(
    pl.kernel,
    out_type=jax.ShapeDtypeStruct((NW, NPAD), jnp.float32),
    mesh=_MESH,
    compiler_params=_SC_PARAMS,
    scratch_types=[
        pltpu.VMEM((EW,), jnp.int32),
        pltpu.VMEM((NPAD,), jnp.float32),
    ],
)(_deg_body)


def _prop_body(hp_hbm, srcs_hbm, dsts_hbm, out_hbm, idxs_v, idxd_v, gbuf, acc,
               semg, sems):
    c = lax.axis_index("c")
    s = lax.axis_index("s")
    wid = s * NC + c
    pltpu.sync_copy(srcs_hbm.at[wid], idxs_v)
    pltpu.sync_copy(dsts_hbm.at[wid], idxd_v)

    zeros = jnp.zeros((L,), jnp.float32)

    @pl.loop(0, K, unroll=4)
    def _(i):
        for j in range(H // L):
            gbuf[0, i, pl.ds(j * L, L)] = zeros

    for t in range(RPT // K):
        pltpu.sync_copy(gbuf.at[0], acc.at[pl.ds(s * RPT + t * K, K)])

    plsc.subcore_barrier()

    def _g(p, base):
        pltpu.async_copy(hp_hbm.at[idxs_v.at[2 * p]], gbuf.at[base], semg)
        pltpu.async_copy(hp_hbm.at[idxs_v.at[2 * p + 1]], gbuf.at[base + 1], semg)

    def _s(p, base):
        pltpu.async_copy(gbuf.at[base], acc.at[idxd_v.at[2 * p]], sems, add=True)
        pltpu.async_copy(
            gbuf.at[base + 1], acc.at[idxd_v.at[2 * p + 1]], sems, add=True
        )

    def _wg():
        pltpu.make_async_copy(hp_hbm.at[pl.ds(0, K)], gbuf.at[0], semg).wait()
        pltpu.make_async_copy(hp_hbm.at[pl.ds(0, K)], gbuf.at[0], semg).wait()

    def _ws():
        pltpu.make_async_copy(gbuf.at[0], acc.at[pl.ds(0, K)], sems).wait()
        pltpu.make_async_copy(gbuf.at[0], acc.at[pl.ds(0, K)], sems).wait()

    NP = CH // 2
    _g(0, 0)
    _g(1, 2)
    _wg()
    _s(0, 0)
    _g(2, 4)
    _wg()
    _s(1, 2)
    _g(3, 6)

    @pl.loop(0, (NP - 4) // 4)
    def _(q):
        for k in range(4):
            p = 4 * q + 2 + k
            slot = (2 + k) % 4
            _wg()
            _s(p, 2 * slot)
            _ws()
            _g(p + 2, 2 * ((slot + 2) % 4))

    _wg()
    _s(NP - 2, 4)
    _ws()
    _wg()
    _s(NP - 1, 6)
    _ws()
    _ws()
    plsc.subcore_barrier()

    pltpu.sync_copy(
        acc.at[pl.ds(s * RPT, RPT)], out_hbm.at[c].at[pl.ds(s * RPT, RPT)]
    )


_prop_call = functools.partial(
    pl.kernel,
    out_type=jax.ShapeDtypeStruct((NC, NPAD, H), jnp.float32),
    mesh=_MESH,
    compiler_params=_SC_PARAMS,
    scratch_types=[
        pltpu.VMEM((CH, K), jnp.int32),
        pltpu.VMEM((CH, K), jnp.int32),
        pltpu.VMEM((8, K, H), jnp.float32),
        pltpu.VMEM_SHARED((NPAD, H), jnp.float32),
        pltpu.SemaphoreType.DMA,
        pltpu.SemaphoreType.DMA,
    ],
)(_prop_body)


def _tc1_body(x_ref, w_ref, degp_ref, hp_ref, dis_ref):
    deg = jnp.sum(degp_ref[...], axis=0) + 1.0
    dis = lax.rsqrt(deg)[:, None]
    h = jnp.dot(x_ref[...], w_ref[...], preferred_element_type=jnp.float32)
    hp_ref[...] = h * dis
    dis_ref[...] = dis


def _tc1_call(xpad, w1, degp):
    return pl.pallas_call(
        _tc1_body,
        grid=(GRID,),
        in_specs=[
            pl.BlockSpec((RB, F_IN), lambda i: (i, 0)),
            pl.BlockSpec((F_IN, H), lambda i: (0, 0)),
            pl.BlockSpec((NW, RB), lambda i: (0, i)),
        ],
        out_specs=[
            pl.BlockSpec((RB, H), lambda i: (i, 0)),
            pl.BlockSpec((RB, 1), lambda i: (i, 0)),
        ],
        out_shape=[
            jax.ShapeDtypeStruct((NPAD, H), jnp.float32),
            jax.ShapeDtypeStruct((NPAD, 1), jnp.float32),
        ],
    )(xpad, w1, degp)


def _tc2_body(accp_ref, hp_ref, dis_ref, b_ref, w_ref, out_ref):
    t = accp_ref[0] + accp_ref[1] + hp_ref[...]
    t = t * dis_ref[...] + b_ref[...]
    t = jnp.maximum(t, 0.0)
    h2 = jnp.dot(t, w_ref[...], preferred_element_type=jnp.float32)
    out_ref[...] = h2 * dis_ref[...]


def _tc2_call(accp, hp, dis, b2d, w):
    return pl.pallas_call(
        _tc2_body,
        grid=(GRID,),
        in_specs=[
            pl.BlockSpec((NC, RB, H), lambda i: (0, i, 0)),
            pl.BlockSpec((RB, H), lambda i: (i, 0)),
            pl.BlockSpec((RB, 1), lambda i: (i, 0)),
            pl.BlockSpec((1, H), lambda i: (0, 0)),
            pl.BlockSpec((H, H), lambda i: (0, 0)),
        ],
        out_specs=pl.BlockSpec((RB, H), lambda i: (i, 0)),
        out_shape=jax.ShapeDtypeStruct((NPAD, H), jnp.float32),
    )(accp, hp, dis, b2d, w)


def _tc3_body(accp_ref, hp_ref, dis_ref, b_ref, out_ref):
    z = accp_ref[0] + accp_ref[1] + hp_ref[...]
    z = z * dis_ref[...] + b_ref[...]
    m = jnp.max(z, axis=1, keepdims=True)
    lse = jnp.log(jnp.sum(jnp.exp(z - m), axis=1, keepdims=True)) + m
    out_ref[...] = z - lse


def _tc3_call(accp, hp, dis, b2d):
    return pl.pallas_call(
        _tc3_body,
        grid=(GRID,),
        in_specs=[
            pl.BlockSpec((NC, RB, H), lambda i: (0, i, 0)),
            pl.BlockSpec((RB, H), lambda i: (i, 0)),
            pl.BlockSpec((RB, 1), lambda i: (i, 0)),
            pl.BlockSpec((1, H), lambda i: (0, 0)),
        ],
        out_specs=pl.BlockSpec((RB, H), lambda i: (i, 0)),
        out_shape=jax.ShapeDtypeStruct((N, H), jnp.float32),
    )(accp, hp, dis, b2d)


def kernel(edges, features, W1, b1, W2, b2, W3, b3):
    src = edges[0].astype(jnp.int32)
    dst = edges[1].astype(jnp.int32)
    pad = jnp.full((NW, EW - EPW), N, jnp.int32)
    srcw = jnp.concatenate([src.reshape(NW, EPW), pad], axis=1)
    dstw = jnp.concatenate([dst.reshape(NW, EPW), pad], axis=1)
    srcs = srcw.reshape(NW, CH, K)
    dsts = dstw.reshape(NW, CH, K)
    degp = _deg_call(dstw)
    xpad = jnp.pad(features, ((0, NPAD - N), (0, 0)))
    hp1, dis = _tc1_call(xpad, W1, degp)
    acc1 = _prop_call(hp1, srcs, dsts)
    hp2 = _tc2_call(acc1, hp1, dis, b1.reshape(1, H), W2)
    acc2 = _prop_call(hp2, srcs, dsts)
    hp3 = _tc2_call(acc2, hp2, dis, b2.reshape(1, H), W3)
    acc3 = _prop_call(hp3, srcs, dsts)
    return _tc3_call(acc3, hp3, dis, b3.reshape(1, H))
